# baseline (speedup 1.0000x reference)
import functools

import jax
import jax.numpy as jnp
from jax import lax
from jax.experimental import pallas as pl
from jax.experimental.pallas import tpu as pltpu
from jax.experimental.pallas import tpu_sc as plsc

_N = 10000
_E = 320000
_D = 128
_CW = 16

_NC = 2
_NS = 16
_NW = _NC * _NS
_EPW = _E // _NW
_CHUNK = 80
_NCH = _EPW // _CHUNK
_SB = 25
_NB = 3
_STRIPE = 624
_TAIL = _N - _NS * _STRIPE


@functools.lru_cache(maxsize=None)
def _make_sc_agg(with_cnt):
    mesh = plsc.VectorSubcoreMesh(core_axis_name="c", subcore_axis_name="s")

    nb = _NB if with_cnt else _NB + 1
    lag = nb - 2

    out_type = [jax.ShapeDtypeStruct((_NC, _N, _D), jnp.float32)]
    scratch = [
        pltpu.VMEM((2, _SB, _CHUNK), jnp.int32),
        pltpu.VMEM((2, _SB, _CHUNK), jnp.int32),
        pltpu.VMEM((nb, _CHUNK, _D), jnp.float32),
        pltpu.VMEM_SHARED((_N, _D), jnp.float32),
        pltpu.SemaphoreType.DMA,
        pltpu.SemaphoreType.DMA,
        pltpu.SemaphoreType.DMA,
    ]
    if with_cnt:
        out_type.append(jax.ShapeDtypeStruct((_NC, _N, _CW), jnp.float32))
        scratch += [
            pltpu.VMEM((_CHUNK, _CW), jnp.float32),
            pltpu.VMEM_SHARED((_N, _CW), jnp.float32),
            pltpu.SemaphoreType.DMA,
        ]

    @functools.partial(
        pl.kernel,
        out_type=tuple(out_type) if with_cnt else out_type[0],
        mesh=mesh,
        scratch_types=scratch,
        compiler_params=pltpu.CompilerParams(use_tc_tiling_on_sc=False),
    )
    def sc_agg(h_hbm, src_hbm, dst_hbm, zero_hbm, *rest):
        if with_cnt:
            (onehot_hbm, zero16_hbm, out_hbm, cnt_hbm,
             src_v, dst_v, rows_v, acc_sh, sem, sem_s, sem_i,
             one_v, cacc_sh, sem_c) = rest
        else:
            (out_hbm, src_v, dst_v, rows_v, acc_sh, sem, sem_s,
             sem_i) = rest

        cid = lax.axis_index("c")
        sid = lax.axis_index("s")
        wid = cid * _NS + sid

        pltpu.sync_copy(zero_hbm.at[pl.ds(sid * _STRIPE, _STRIPE)],
                        acc_sh.at[pl.ds(sid * _STRIPE, _STRIPE)])
        if with_cnt:
            pltpu.sync_copy(zero16_hbm.at[pl.ds(sid * _STRIPE, _STRIPE)],
                            cacc_sh.at[pl.ds(sid * _STRIPE, _STRIPE)])
            pltpu.sync_copy(onehot_hbm, one_v)

        @pl.when(sid == 0)
        def _():
            pltpu.sync_copy(zero_hbm.at[pl.ds(_NS * _STRIPE, _TAIL)],
                            acc_sh.at[pl.ds(_NS * _STRIPE, _TAIL)])
            if with_cnt:
                pltpu.sync_copy(zero16_hbm.at[pl.ds(_NS * _STRIPE, _TAIL)],
                                cacc_sh.at[pl.ds(_NS * _STRIPE, _TAIL)])

        pltpu.sync_copy(src_hbm.at[wid, pl.ds(0, _SB)], src_v.at[0])
        pltpu.sync_copy(dst_hbm.at[wid, pl.ds(0, _SB)], dst_v.at[0])
        for p in range(2):
            pltpu.async_copy(h_hbm.at[src_v.at[0, p]], rows_v.at[p], sem)

        plsc.subcore_barrier()

        def body(i, c):
            r = lax.rem(i, _SB)
            sbuf = lax.rem(lax.div(i, _SB), 2)
            buf = lax.rem(i, nb)

            pltpu.make_async_copy(h_hbm.at[src_v.at[sbuf, r]],
                                  rows_v.at[buf], sem).wait()
            pltpu.async_copy(rows_v.at[buf], acc_sh.at[dst_v.at[sbuf, r]],
                             sem_s, add=True)
            if with_cnt:
                pltpu.async_copy(one_v, cacc_sh.at[dst_v.at[sbuf, r]],
                                 sem_c, add=True)

            @pl.when(i >= lag)
            def _():
                pltpu.make_async_copy(rows_v.at[buf],
                                      acc_sh.at[dst_v.at[sbuf, r]],
                                      sem_s).wait()
                if with_cnt:
                    pltpu.make_async_copy(one_v, cacc_sh.at[dst_v.at[sbuf, r]],
                                          sem_c).wait()

            @pl.when((r == 0) & (i + _SB < _NCH))
            def _():
                nb = 1 - sbuf
                pltpu.async_copy(src_hbm.at[wid, pl.ds(i + _SB, _SB)],
                                 src_v.at[nb], sem_i)
                pltpu.async_copy(dst_hbm.at[wid, pl.ds(i + _SB, _SB)],
                                 dst_v.at[nb], sem_i)

            @pl.when((r == _SB - 2) & (i + 2 < _NCH))
            def _():
                pltpu.make_async_copy(src_hbm.at[wid, pl.ds(0, _SB)],
                                      src_v.at[0], sem_i).wait()
                pltpu.make_async_copy(dst_hbm.at[wid, pl.ds(0, _SB)],
                                      dst_v.at[0], sem_i).wait()

            @pl.when(i + 2 < _NCH)
            def _():
                j = i + 2
                rj = lax.rem(j, _SB)
                sj = lax.rem(lax.div(j, _SB), 2)
                pltpu.async_copy(h_hbm.at[src_v.at[sj, rj]],
                                 rows_v.at[lax.rem(j, nb)], sem)

            return c

        lax.fori_loop(0, _NCH, body, 0)

        for _ in range(lag):
            pltpu.make_async_copy(rows_v.at[0], acc_sh.at[dst_v.at[0, 0]],
                                  sem_s).wait()
        if with_cnt:
            pltpu.make_async_copy(one_v, cacc_sh.at[dst_v.at[0, 0]],
                                  sem_c).wait()

        plsc.subcore_barrier()

        pltpu.sync_copy(acc_sh.at[pl.ds(sid * _STRIPE, _STRIPE)],
                        out_hbm.at[cid, pl.ds(sid * _STRIPE, _STRIPE)])
        if with_cnt:
            pltpu.sync_copy(cacc_sh.at[pl.ds(sid * _STRIPE, _STRIPE)],
                            cnt_hbm.at[cid, pl.ds(sid * _STRIPE, _STRIPE)])

        @pl.when(sid == 0)
        def _():
            pltpu.sync_copy(acc_sh.at[pl.ds(_NS * _STRIPE, _TAIL)],
                            out_hbm.at[cid, pl.ds(_NS * _STRIPE, _TAIL)])
            if with_cnt:
                pltpu.sync_copy(cacc_sh.at[pl.ds(_NS * _STRIPE, _TAIL)],
                                cnt_hbm.at[cid, pl.ds(_NS * _STRIPE, _TAIL)])

    return sc_agg


def _dense1_body(acc_ref, cacc_ref, x_ref, wl_ref, bl_ref, wr_ref, g_ref,
                 be_ref, h_ref, cnt_ref):
    s = acc_ref[0] + acc_ref[1]
    cnt = cacc_ref[0, :, 0:1] + cacc_ref[1, :, 0:1]
    mean = s / jnp.maximum(cnt, 1.0)
    z = (lax.dot_general(mean, wl_ref[...], (((1,), (1,)), ((), ())),
                         preferred_element_type=jnp.float32)
         + lax.dot_general(x_ref[...], wr_ref[...], (((1,), (1,)), ((), ())),
                           preferred_element_type=jnp.float32)
         + bl_ref[...])
    mu = jnp.mean(z, axis=0)
    var = jnp.mean((z - mu) ** 2, axis=0)
    zn = (z - mu) / jnp.sqrt(var + 1e-5) * g_ref[...] + be_ref[...]
    h_ref[...] = jnp.maximum(zn, 0.0)
    cnt_ref[...] = cnt


def _dense2_body(acc_ref, cnt_ref, h_ref, wl_ref, bl_ref, wr_ref, g_ref,
                 be_ref, wc_ref, bc_ref, out_ref):
    s = acc_ref[0] + acc_ref[1]
    mean = s / jnp.maximum(cnt_ref[...], 1.0)
    z = (lax.dot_general(mean, wl_ref[...], (((1,), (1,)), ((), ())),
                         preferred_element_type=jnp.float32)
         + lax.dot_general(h_ref[...], wr_ref[...], (((1,), (1,)), ((), ())),
                           preferred_element_type=jnp.float32)
         + bl_ref[...])
    mu = jnp.mean(z, axis=0)
    var = jnp.mean((z - mu) ** 2, axis=0)
    zn = (z - mu) / jnp.sqrt(var + 1e-5) * g_ref[...] + be_ref[...]
    h2 = jnp.maximum(zn, 0.0)
    logits = (lax.dot_general(h2, wc_ref[...], (((1,), (1,)), ((), ())),
                              preferred_element_type=jnp.float32)
              + bc_ref[...])
    out_ref[...] = logits


def kernel(x, edge_index, W1l, b1l, W1r, g1, be1, W2l, b2l, W2r, g2, be2,
           Wc, bc):
    src = edge_index[0].reshape(_NW, _NCH, _CHUNK)
    dst = edge_index[1].reshape(_NW, _NCH, _CHUNK)

    zero_d = jnp.zeros((_N, _D), jnp.float32)
    zero16 = jnp.zeros((_N, _CW), jnp.float32)
    onehot = jnp.zeros((_CHUNK, _CW), jnp.float32).at[:, 0].set(1.0)

    acc1, cacc = _make_sc_agg(True)(x, src, dst, zero_d, onehot, zero16)

    h1, cnt = pl.pallas_call(
        _dense1_body,
        out_shape=(jax.ShapeDtypeStruct((_N, _D), jnp.float32),
                   jax.ShapeDtypeStruct((_N, 1), jnp.float32)),
    )(acc1, cacc, x, W1l, b1l, W1r, g1, be1)

    acc2 = _make_sc_agg(False)(h1, src, dst, zero_d)

    return pl.pallas_call(
        _dense2_body,
        out_shape=jax.ShapeDtypeStruct((_N, Wc.shape[0]), jnp.float32),
    )(acc2, cnt, h1, W2l, b2l, W2r, g2, be2, Wc, bc)


# speedup vs baseline: 1.0106x; 1.0106x over previous
import functools

import jax
import jax.numpy as jnp
from jax import lax
from jax.experimental import pallas as pl
from jax.experimental.pallas import tpu as pltpu
from jax.experimental.pallas import tpu_sc as plsc

_N = 10000
_E = 320000
_D = 128
_CW = 16

_NC = 2
_NS = 16
_NW = _NC * _NS
_EPW = _E // _NW
_CHUNK = 80
_NCH = _EPW // _CHUNK
_SB = 25
_NB = 3
_STRIPE = 624
_TAIL = _N - _NS * _STRIPE


@functools.lru_cache(maxsize=None)
def _make_sc_agg(with_cnt):
    mesh = plsc.VectorSubcoreMesh(core_axis_name="c", subcore_axis_name="s")

    nb = _NB
    lag = nb - 2

    out_type = [jax.ShapeDtypeStruct((_NC, _N, _D), jnp.float32)]
    scratch = [
        pltpu.VMEM((2, _SB, _CHUNK), jnp.int32),
        pltpu.VMEM((2, _SB, _CHUNK), jnp.int32),
        pltpu.VMEM((nb, _CHUNK, _D), jnp.float32),
        pltpu.VMEM_SHARED((_N, _D), jnp.float32),
        pltpu.SemaphoreType.DMA,
        pltpu.SemaphoreType.DMA,
        pltpu.SemaphoreType.DMA,
    ]
    if with_cnt:
        out_type.append(jax.ShapeDtypeStruct((_NC, _N, _CW), jnp.float32))
        scratch += [
            pltpu.VMEM((_CHUNK, _CW), jnp.float32),
            pltpu.VMEM_SHARED((_N, _CW), jnp.float32),
            pltpu.SemaphoreType.DMA,
        ]

    @functools.partial(
        pl.kernel,
        out_type=tuple(out_type) if with_cnt else out_type[0],
        mesh=mesh,
        scratch_types=scratch,
        compiler_params=pltpu.CompilerParams(use_tc_tiling_on_sc=False),
    )
    def sc_agg(h_hbm, src_hbm, dst_hbm, zero_hbm, *rest):
        if with_cnt:
            (onehot_hbm, zero16_hbm, out_hbm, cnt_hbm,
             src_v, dst_v, rows_v, acc_sh, sem, sem_s, sem_i,
             one_v, cacc_sh, sem_c) = rest
        else:
            (out_hbm, src_v, dst_v, rows_v, acc_sh, sem, sem_s,
             sem_i) = rest

        cid = lax.axis_index("c")
        sid = lax.axis_index("s")
        wid = cid * _NS + sid

        pltpu.sync_copy(zero_hbm.at[pl.ds(sid * _STRIPE, _STRIPE)],
                        acc_sh.at[pl.ds(sid * _STRIPE, _STRIPE)])
        if with_cnt:
            pltpu.sync_copy(zero16_hbm.at[pl.ds(sid * _STRIPE, _STRIPE)],
                            cacc_sh.at[pl.ds(sid * _STRIPE, _STRIPE)])
            pltpu.sync_copy(onehot_hbm, one_v)

        @pl.when(sid == 0)
        def _():
            pltpu.sync_copy(zero_hbm.at[pl.ds(_NS * _STRIPE, _TAIL)],
                            acc_sh.at[pl.ds(_NS * _STRIPE, _TAIL)])
            if with_cnt:
                pltpu.sync_copy(zero16_hbm.at[pl.ds(_NS * _STRIPE, _TAIL)],
                                cacc_sh.at[pl.ds(_NS * _STRIPE, _TAIL)])

        pltpu.sync_copy(src_hbm.at[wid, pl.ds(0, _SB)], src_v.at[0])
        pltpu.sync_copy(dst_hbm.at[wid, pl.ds(0, _SB)], dst_v.at[0])
        for p in range(2):
            pltpu.async_copy(h_hbm.at[src_v.at[0, p]], rows_v.at[p], sem)

        plsc.subcore_barrier()

        def body(i, c):
            r = lax.rem(i, _SB)
            sbuf = lax.rem(lax.div(i, _SB), 2)
            buf = lax.rem(i, nb)

            pltpu.make_async_copy(h_hbm.at[src_v.at[sbuf, r]],
                                  rows_v.at[buf], sem).wait()
            pltpu.async_copy(rows_v.at[buf], acc_sh.at[dst_v.at[sbuf, r]],
                             sem_s, add=True)
            if with_cnt:
                pltpu.async_copy(one_v, cacc_sh.at[dst_v.at[sbuf, r]],
                                 sem_c, add=True)

            @pl.when(i >= lag)
            def _():
                pltpu.make_async_copy(rows_v.at[buf],
                                      acc_sh.at[dst_v.at[sbuf, r]],
                                      sem_s).wait()
                if with_cnt:
                    pltpu.make_async_copy(one_v, cacc_sh.at[dst_v.at[sbuf, r]],
                                          sem_c).wait()

            @pl.when((r == 0) & (i + _SB < _NCH))
            def _():
                nb = 1 - sbuf
                pltpu.async_copy(src_hbm.at[wid, pl.ds(i + _SB, _SB)],
                                 src_v.at[nb], sem_i)
                pltpu.async_copy(dst_hbm.at[wid, pl.ds(i + _SB, _SB)],
                                 dst_v.at[nb], sem_i)

            @pl.when((r == _SB - 2) & (i + 2 < _NCH))
            def _():
                pltpu.make_async_copy(src_hbm.at[wid, pl.ds(0, _SB)],
                                      src_v.at[0], sem_i).wait()
                pltpu.make_async_copy(dst_hbm.at[wid, pl.ds(0, _SB)],
                                      dst_v.at[0], sem_i).wait()

            @pl.when(i + 2 < _NCH)
            def _():
                j = i + 2
                rj = lax.rem(j, _SB)
                sj = lax.rem(lax.div(j, _SB), 2)
                pltpu.async_copy(h_hbm.at[src_v.at[sj, rj]],
                                 rows_v.at[lax.rem(j, nb)], sem)

            return c

        lax.fori_loop(0, _NCH, body, 0)

        for _ in range(lag):
            pltpu.make_async_copy(rows_v.at[0], acc_sh.at[dst_v.at[0, 0]],
                                  sem_s).wait()
        if with_cnt:
            pltpu.make_async_copy(one_v, cacc_sh.at[dst_v.at[0, 0]],
                                  sem_c).wait()

        plsc.subcore_barrier()

        pltpu.sync_copy(acc_sh.at[pl.ds(sid * _STRIPE, _STRIPE)],
                        out_hbm.at[cid, pl.ds(sid * _STRIPE, _STRIPE)])
        if with_cnt:
            pltpu.sync_copy(cacc_sh.at[pl.ds(sid * _STRIPE, _STRIPE)],
                            cnt_hbm.at[cid, pl.ds(sid * _STRIPE, _STRIPE)])

        @pl.when(sid == 0)
        def _():
            pltpu.sync_copy(acc_sh.at[pl.ds(_NS * _STRIPE, _TAIL)],
                            out_hbm.at[cid, pl.ds(_NS * _STRIPE, _TAIL)])
            if with_cnt:
                pltpu.sync_copy(cacc_sh.at[pl.ds(_NS * _STRIPE, _TAIL)],
                                cnt_hbm.at[cid, pl.ds(_NS * _STRIPE, _TAIL)])

    return sc_agg


def _dense1_body(acc_ref, cacc_ref, x_ref, wl_ref, bl_ref, wr_ref, g_ref,
                 be_ref, h_ref, cnt_ref):
    s = acc_ref[0] + acc_ref[1]
    cnt = cacc_ref[0, :, 0:1] + cacc_ref[1, :, 0:1]
    mean = s / jnp.maximum(cnt, 1.0)
    z = (lax.dot_general(mean, wl_ref[...], (((1,), (1,)), ((), ())),
                         preferred_element_type=jnp.float32)
         + lax.dot_general(x_ref[...], wr_ref[...], (((1,), (1,)), ((), ())),
                           preferred_element_type=jnp.float32)
         + bl_ref[...])
    mu = jnp.mean(z, axis=0)
    var = jnp.mean((z - mu) ** 2, axis=0)
    zn = (z - mu) / jnp.sqrt(var + 1e-5) * g_ref[...] + be_ref[...]
    h_ref[...] = jnp.maximum(zn, 0.0)
    cnt_ref[...] = cnt


def _dense2_body(acc_ref, cnt_ref, h_ref, wl_ref, bl_ref, wr_ref, g_ref,
                 be_ref, wc_ref, bc_ref, out_ref):
    s = acc_ref[0] + acc_ref[1]
    mean = s / jnp.maximum(cnt_ref[...], 1.0)
    z = (lax.dot_general(mean, wl_ref[...], (((1,), (1,)), ((), ())),
                         preferred_element_type=jnp.float32)
         + lax.dot_general(h_ref[...], wr_ref[...], (((1,), (1,)), ((), ())),
                           preferred_element_type=jnp.float32)
         + bl_ref[...])
    mu = jnp.mean(z, axis=0)
    var = jnp.mean((z - mu) ** 2, axis=0)
    zn = (z - mu) / jnp.sqrt(var + 1e-5) * g_ref[...] + be_ref[...]
    h2 = jnp.maximum(zn, 0.0)
    logits = (lax.dot_general(h2, wc_ref[...], (((1,), (1,)), ((), ())),
                              preferred_element_type=jnp.float32)
              + bc_ref[...])
    out_ref[...] = logits


def kernel(x, edge_index, W1l, b1l, W1r, g1, be1, W2l, b2l, W2r, g2, be2,
           Wc, bc):
    src = edge_index[0].reshape(_NW, _NCH, _CHUNK)
    dst = edge_index[1].reshape(_NW, _NCH, _CHUNK)

    zero_d = jnp.zeros((_N, _D), jnp.float32)
    zero16 = jnp.zeros((_N, _CW), jnp.float32)
    onehot = jnp.zeros((_CHUNK, _CW), jnp.float32).at[:, 0].set(1.0)

    acc1, cacc = _make_sc_agg(True)(x, src, dst, zero_d, onehot, zero16)

    h1, cnt = pl.pallas_call(
        _dense1_body,
        out_shape=(jax.ShapeDtypeStruct((_N, _D), jnp.float32),
                   jax.ShapeDtypeStruct((_N, 1), jnp.float32)),
    )(acc1, cacc, x, W1l, b1l, W1r, g1, be1)

    acc2 = _make_sc_agg(False)(h1, src, dst, zero_d)

    return pl.pallas_call(
        _dense2_body,
        out_shape=jax.ShapeDtypeStruct((_N, Wc.shape[0]), jnp.float32),
    )(acc2, cnt, h1, W2l, b2l, W2r, g2, be2, Wc, bc)
